# baseline (speedup 1.0000x reference)
import functools

import jax
import jax.numpy as jnp
from jax import lax
from jax.experimental import pallas as pl
from jax.experimental.pallas import tpu as pltpu
from jax.experimental.pallas import tpu_sc as plsc


N = 4096
D = 1024
B = 4
L = 16
NW = 32
RPW = N // NW
CH = 16
NCHUNK = RPW // CH
NITEM = NCHUNK * B


def _sc_body(x_hbm, emb_hbm, pos_hbm, out_hbm,
             idx_v, e0, e1, x0, x1, x2, x3,
             sg0, sg1, si0, si1, si2, si3, so0, so1, so2, so3):
    nc = plsc.get_sparse_core_info().num_cores
    wid = lax.axis_index("s") * nc + lax.axis_index("c")
    base = wid * RPW

    emb_bufs = [e0, e1]
    x_bufs = [x0, x1, x2, x3]
    sg = [sg0, sg1]
    si = [si0, si1, si2, si3]
    so = [so0, so1, so2, so3]

    def start_gather(c):
        s = c % 2
        cp = pltpu.async_copy(
            emb_hbm.at[idx_v.at[pl.ds(c * CH, CH)]], emb_bufs[s], sg[s])
        return cp

    def start_in(i):
        c, b = divmod(i, B)
        q = i % 4
        cp = pltpu.async_copy(
            x_hbm.at[pl.ds(b * N + base + c * CH, CH)], x_bufs[q], si[q])
        return cp

    def start_out(i):
        c, b = divmod(i, B)
        q = i % 4
        cp = pltpu.async_copy(
            x_bufs[q], out_hbm.at[pl.ds(b * N + base + c * CH, CH)], so[q])
        return cp

    HALF = D // 2

    def add_rows(xb, eb):
        @plsc.parallel_loop(0, CH * 2)
        def _(t):
            r = t // 2
            col = (t % 2) * HALF
            for j in range(HALF // L):
                plsc.addupdate(xb.at[r, pl.ds(col + j * L, L)],
                               eb[r, pl.ds(col + j * L, L)])

    pend_i = {i: start_in(i) for i in range(3)}
    pltpu.sync_copy(pos_hbm.at[pl.ds(base, RPW)], idx_v)
    pend_g = {0: start_gather(0)}
    pend_o = {}

    for i in range(NITEM):
        c, b = divmod(i, B)
        q = i % 4
        if b == 0:
            if c + 1 < NCHUNK:
                pend_g[c + 1] = start_gather(c + 1)
            pend_g.pop(c).wait()
        pend_i.pop(i).wait()
        add_rows(x_bufs[q], emb_bufs[c % 2])
        pend_o[i] = start_out(i)
        j = i + 3
        if j < NITEM:
            if i >= 1:
                pend_o.pop(i - 1).wait()
            pend_i[j] = start_in(j)

    for i in sorted(pend_o):
        pend_o.pop(i).wait()


_sc_call = functools.partial(
    pl.kernel,
    mesh=plsc.VectorSubcoreMesh(core_axis_name="c", subcore_axis_name="s"),
    out_type=jax.ShapeDtypeStruct((B * N, D), jnp.float32),
    scratch_types=(
        [pltpu.VMEM((RPW,), jnp.int32)]
        + [pltpu.VMEM((CH, D), jnp.float32)] * 2
        + [pltpu.VMEM((CH, D), jnp.float32)] * 4
        + [pltpu.SemaphoreType.DMA] * 10
    ),
)(_sc_body)


def kernel(x, positional_embedding, positions):
    x2d = x.reshape(B * N, D)
    pos32 = positions.astype(jnp.int32)
    out2d = _sc_call(x2d, positional_embedding, pos32)
    return out2d.reshape(B, N, D)


# speedup vs baseline: 1.0437x; 1.0437x over previous
import functools

import jax
import jax.numpy as jnp
from jax import lax
from jax.experimental import pallas as pl
from jax.experimental.pallas import tpu as pltpu
from jax.experimental.pallas import tpu_sc as plsc


N = 4096
D = 1024
B = 4
L = 16
NW = 32
RPW = N // NW
CH = 16
NCHUNK = RPW // CH
NITEM = NCHUNK * B


def _sc_body(x_hbm, emb_hbm, pos_hbm, out_hbm,
             idx_v, e0, e1, x0, x1, x2, x3,
             sg0, sg1, si0, si1, si2, si3, so0, so1, so2, so3):
    nc = plsc.get_sparse_core_info().num_cores
    wid = lax.axis_index("s") * nc + lax.axis_index("c")
    base = wid * RPW

    emb_bufs = [e0, e1]
    x_bufs = [x0, x1, x2, x3]
    sg = [sg0, sg1]
    si = [si0, si1, si2, si3]
    so = [so0, so1, so2, so3]

    def start_gather(c):
        s = c % 2
        cp = pltpu.async_copy(
            emb_hbm.at[idx_v.at[pl.ds(c * CH, CH)]], emb_bufs[s], sg[s])
        return cp

    def start_in(i):
        c, b = divmod(i, B)
        q = i % 4
        cp = pltpu.async_copy(
            x_hbm.at[pl.ds(b * N + base + c * CH, CH)], x_bufs[q], si[q])
        return cp

    def start_out(i):
        c, b = divmod(i, B)
        q = i % 4
        cp = pltpu.async_copy(
            x_bufs[q], out_hbm.at[pl.ds(b * N + base + c * CH, CH)], so[q])
        return cp

    QTR = D // 4

    def add_rows(xb, eb):
        @plsc.parallel_loop(0, CH * 4)
        def _(t):
            r = t // 4
            col = (t % 4) * QTR
            for j in range(QTR // L):
                plsc.addupdate(xb.at[r, pl.ds(col + j * L, L)],
                               eb[r, pl.ds(col + j * L, L)])

    pend_i = {i: start_in(i) for i in range(3)}
    pltpu.sync_copy(pos_hbm.at[pl.ds(base, RPW)], idx_v)
    pend_g = {0: start_gather(0)}
    pend_o = {}

    for i in range(NITEM):
        c, b = divmod(i, B)
        q = i % 4
        if b == 0:
            if c + 1 < NCHUNK:
                pend_g[c + 1] = start_gather(c + 1)
            pend_g.pop(c).wait()
        pend_i.pop(i).wait()
        add_rows(x_bufs[q], emb_bufs[c % 2])
        pend_o[i] = start_out(i)
        j = i + 3
        if j < NITEM:
            if i >= 1:
                pend_o.pop(i - 1).wait()
            pend_i[j] = start_in(j)

    for i in sorted(pend_o):
        pend_o.pop(i).wait()


_sc_call = functools.partial(
    pl.kernel,
    mesh=plsc.VectorSubcoreMesh(core_axis_name="c", subcore_axis_name="s"),
    out_type=jax.ShapeDtypeStruct((B * N, D), jnp.float32),
    scratch_types=(
        [pltpu.VMEM((RPW,), jnp.int32)]
        + [pltpu.VMEM((CH, D), jnp.float32)] * 2
        + [pltpu.VMEM((CH, D), jnp.float32)] * 4
        + [pltpu.SemaphoreType.DMA] * 10
    ),
)(_sc_body)


def kernel(x, positional_embedding, positions):
    x2d = x.reshape(B * N, D)
    pos32 = positions.astype(jnp.int32)
    out2d = _sc_call(x2d, positional_embedding, pos32)
    return out2d.reshape(B, N, D)
